# baseline (speedup 1.0000x reference)
import functools

import jax
import jax.numpy as jnp
from jax import lax
from jax.experimental import pallas as pl
from jax.experimental.pallas import tpu as pltpu
from jax.experimental.pallas import tpu_sc as plsc

DIM = 64
B = 16384
LAM = 0.001

NC = 2
NS = 16
NW = NC * NS

SSPAN = 31232
W = 256
NPAIR = 61
XBASE = 999424
TAIL0 = 999936
SCAP = 144
DUMP = B

_i32 = jnp.int32
_f32 = jnp.float32


def _lane():
    return lax.iota(_i32, 16)


def _splat(x, dtype=_i32):
    return jnp.full((16,), x, dtype)


def _bin_ids(ids_hbm, idbuf, binref, lo, hi):
    lane = _lane()

    def chunk(ci, cnt):
        pltpu.sync_copy(ids_hbm.at[pl.ds(ci * 2048, 2048)], idbuf)

        def vec(jv, cnt):
            v = idbuf[pl.ds(jv * 16, 16)]
            mask = (v >= lo) & (v < hi)
            pos = ci * 2048 + jv * 16 + lane
            packed = ((v - lo) << 14) | pos
            cum = plsc.cumsum(mask.astype(_i32))
            dest = cnt + cum - 1
            plsc.store_scatter(binref, [dest], packed, mask=mask)
            return cnt + plsc.all_reduce_population_count(mask)[0]

        return lax.fori_loop(0, 128, vec, cnt)

    return lax.fori_loop(0, 8, chunk, jnp.int32(0))


def _scan_chunk(buf, wc, c0l, binref, cnt, staged, posbuf, inter, semS, m):
    lane = _lane()
    c0 = jnp.minimum(jnp.asarray(c0l, _i32), 1 << 15)
    lo_p = c0 << 14
    hi_p = (c0 + wc) << 14
    nv = (cnt + 15) >> 4

    def vec(jv, m):
        idx16 = jv * 16 + lane
        v = binref[pl.ds(jv * 16, 16)]
        inr = (v >= lo_p) & (v < hi_p) & (idx16 < cnt)
        pci = plsc.all_reduce_population_count(inr)[0]
        cum = plsc.cumsum(inr.astype(_i32))
        dest = m + cum - 1
        cols = (v >> 14) - c0
        pos = v & 0x3FFF

        @pl.when(pci > 0)
        def _():
            def feat(j, carry):
                js = _splat(0) + j
                val = plsc.load_gather(buf, [js, cols], mask=inr)
                plsc.store_scatter(staged, [dest, js], val, mask=inr)
                return carry

            lax.fori_loop(0, DIM, feat, 0)
            plsc.store_scatter(posbuf, [dest >> 7, dest & 127], pos,
                               mask=inr)

        m2 = m + pci

        @pl.when(m2 >= 128)
        def _():
            pltpu.async_copy(staged.at[pl.ds(0, 128)],
                             inter.at[posbuf.at[0]], semS).wait()
            src = 128 + lane

            def feat(j, carry):
                js = _splat(0) + j
                sp = plsc.load_gather(staged, [src, js])
                plsc.store_scatter(staged, [lane, js], sp)
                return carry

            lax.fori_loop(0, DIM, feat, 0)
            pp = plsc.load_gather(posbuf, [_splat(1), lane])
            plsc.store_scatter(posbuf, [_splat(0), lane], pp)

        return jnp.where(m2 >= 128, m2 - 128, m2)

    return lax.fori_loop(0, nv, vec, m)


def _final_flush(staged, posbuf, inter, semS, m):
    lane = _lane()
    for v8 in range(8):
        idxv = v8 * 16 + lane
        plsc.store_scatter(posbuf, [_splat(0), idxv], _splat(DUMP),
                           mask=idxv >= m)
    pltpu.async_copy(staged.at[pl.ds(0, 128)],
                     inter.at[posbuf.at[0]], semS).wait()


def _sc1_body(ue_hbm, ie_hbm, tu_hbm, ti_hbm, uid_hbm, pid_hbm, nid_hbm,
              iu_hbm, ip_hbm, in_hbm,
              idbuf, ubin, pbin, nbin,
              buf0, buf1, tailbuf, stA, stB, pbA, pbB,
              sem0, sem1, semS):
    c = lax.axis_index("c")
    s = lax.axis_index("s")
    wid = s * NC + c
    lo = wid * SSPAN
    hi = jnp.where(wid == NW - 1, 1000000, lo + SSPAN)

    cu = _bin_ids(uid_hbm, idbuf, ubin, lo, hi)
    cp_ = _bin_ids(pid_hbm, idbuf, pbin, lo, hi)
    cn = _bin_ids(nid_hbm, idbuf, nbin, lo, hi)

    def sweep(table, tail, lists, ms):
        def scan_all(buf, wc, c0l, ms):
            return tuple(
                _scan_chunk(buf, wc, c0l, b, ct, st, pb, it, semS, m)
                for (b, ct, st, pb, it), m in zip(lists, ms))

        def pair(kp, ms):
            e0a = lo + (2 * kp) * W
            e0b = lo + (2 * kp + 1) * W
            cpa = pltpu.async_copy(table.at[:, pl.ds(e0a, W)], buf0, sem0)
            cpb = pltpu.async_copy(table.at[:, pl.ds(e0b, W)], buf1, sem1)
            cpa.wait()
            ms = scan_all(buf0, W, (2 * kp) * W, ms)
            cpb.wait()
            ms = scan_all(buf1, W, (2 * kp + 1) * W, ms)
            return ms

        ms = lax.fori_loop(0, NPAIR, pair, tuple(ms))

        def extra(k, ms):
            e0 = XBASE + k * W
            pltpu.sync_copy(table.at[:, pl.ds(e0, W)], buf0)
            return scan_all(buf0, W, e0 - lo, ms)

        ms = lax.fori_loop(0, 2, extra, ms)
        pltpu.sync_copy(tail, tailbuf)
        ms = scan_all(tailbuf, 64, jnp.int32(TAIL0) - lo, ms)
        return ms

    z = jnp.int32(0)
    (mu,) = sweep(ue_hbm, tu_hbm, [(ubin, cu, stA, pbA, iu_hbm)], (z,))
    _final_flush(stA, pbA, iu_hbm, semS, mu)
    mp, mn = sweep(ie_hbm, ti_hbm, [(pbin, cp_, stA, pbA, ip_hbm),
                                    (nbin, cn, stB, pbB, in_hbm)], (z, z))
    _final_flush(stA, pbA, ip_hbm, semS, mp)
    _final_flush(stB, pbB, in_hbm, semS, mn)


HALF = 256


def _sc2_body(iu_hbm, ip_hbm, in_hbm, diff_hbm, l2_hbm,
              ubuf, pbuf, nbuf, diff_v, l2_v, sem):
    c = lax.axis_index("c")
    s = lax.axis_index("s")
    wid = s * NC + c

    lane = _lane()
    zero = jnp.zeros((16,), _f32)

    for h in range(2):
        base = wid * (2 * HALF) + h * HALF
        cps = [pltpu.async_copy(iu_hbm.at[pl.ds(base, HALF)], ubuf, sem),
               pltpu.async_copy(ip_hbm.at[pl.ds(base, HALF)], pbuf, sem),
               pltpu.async_copy(in_hbm.at[pl.ds(base, HALF)], nbuf, sem)]
        for cp in cps:
            cp.wait()

        def group(g, carry):
            bvec = g * 16 + lane

            def feat(j, acc):
                pos, neg, l2 = acc
                js = _splat(0) + j
                u = plsc.load_gather(ubuf, [bvec, js])
                p = plsc.load_gather(pbuf, [bvec, js])
                n = plsc.load_gather(nbuf, [bvec, js])
                return (pos + u * p, neg + u * n,
                        l2 + (u * u + (p * p + n * n)))

            pos, neg, l2 = lax.fori_loop(0, DIM, feat, (zero, zero, zero))
            off = h * HALF + g * 16
            diff_v[pl.ds(off, 16)] = neg - pos
            l2_v[pl.ds(off, 16)] = 0.5 * l2
            return carry

        lax.fori_loop(0, HALF // 16, group, 0)

    pltpu.sync_copy(diff_v, diff_hbm.at[pl.ds(wid * 2 * HALF, 2 * HALF)])
    pltpu.sync_copy(l2_v, l2_hbm.at[pl.ds(wid * 2 * HALF, 2 * HALF)])


def _tc_tail_body(ue_ref, ie_ref, tu_ref, ti_ref):
    tu_ref[...] = ue_ref[...]
    ti_ref[...] = ie_ref[...]


def _tc_body(diff_ref, l2_ref, out_ref):
    x = diff_ref[:]
    sp = jnp.maximum(x, 0.0) + jnp.log1p(jnp.exp(-jnp.abs(x)))
    out_ref[0, 0] = jnp.sum(sp) / B + LAM * (jnp.sum(l2_ref[:]) / B)


def kernel(user_embed, item_embed, user_ids, item_pos_ids, item_neg_ids):
    uid = user_ids.astype(_i32)
    pid = item_pos_ids.astype(_i32)
    nid = item_neg_ids.astype(_i32)

    mesh = plsc.VectorSubcoreMesh(core_axis_name="c", subcore_axis_name="s")
    params = pltpu.CompilerParams(needs_layout_passes=False)

    ueT = user_embed.T
    ieT = item_embed.T
    tail_spec = pl.BlockSpec((DIM, 128), lambda i: (0, TAIL0 // 128))
    out_spec = pl.BlockSpec((DIM, 128), lambda i: (0, 0))
    tu, ti = pl.pallas_call(
        _tc_tail_body,
        grid=(1,),
        out_shape=[jax.ShapeDtypeStruct((DIM, 128), _f32),
                   jax.ShapeDtypeStruct((DIM, 128), _f32)],
        in_specs=[tail_spec, tail_spec],
        out_specs=[out_spec, out_spec],
    )(ueT, ieT)

    sc1 = functools.partial(
        pl.kernel,
        mesh=mesh,
        compiler_params=params,
        out_type=[
            jax.ShapeDtypeStruct((B + 16, 128), _f32),
            jax.ShapeDtypeStruct((B + 16, 128), _f32),
            jax.ShapeDtypeStruct((B + 16, 128), _f32),
        ],
        scratch_types=[
            pltpu.VMEM((2048,), _i32),
            pltpu.VMEM((B,), _i32),
            pltpu.VMEM((B,), _i32),
            pltpu.VMEM((B,), _i32),
            pltpu.VMEM((DIM, W), _f32),
            pltpu.VMEM((DIM, W), _f32),
            pltpu.VMEM((DIM, 128), _f32),
            pltpu.VMEM((SCAP, 128), _f32),
            pltpu.VMEM((SCAP, 128), _f32),
            pltpu.VMEM((2, 128), _i32),
            pltpu.VMEM((2, 128), _i32),
            pltpu.SemaphoreType.DMA,
            pltpu.SemaphoreType.DMA,
            pltpu.SemaphoreType.DMA,
        ],
    )(_sc1_body)
    iu, ip_, in_ = sc1(ueT, ieT, tu, ti, uid, pid, nid)

    sc2 = functools.partial(
        pl.kernel,
        mesh=mesh,
        compiler_params=params,
        out_type=[
            jax.ShapeDtypeStruct((B,), _f32),
            jax.ShapeDtypeStruct((B,), _f32),
        ],
        scratch_types=[
            pltpu.VMEM((HALF, 128), _f32),
            pltpu.VMEM((HALF, 128), _f32),
            pltpu.VMEM((HALF, 128), _f32),
            pltpu.VMEM((2 * HALF,), _f32),
            pltpu.VMEM((2 * HALF,), _f32),
            pltpu.SemaphoreType.DMA,
        ],
    )(_sc2_body)
    diff, l2row = sc2(iu, ip_, in_)

    out = pl.pallas_call(
        _tc_body,
        out_shape=jax.ShapeDtypeStruct((1, 1), _f32),
        out_specs=pl.BlockSpec(memory_space=pltpu.SMEM),
    )(diff.reshape(B // 128, 128), l2row.reshape(B // 128, 128))
    return out[0, 0]


# speedup vs baseline: 1.3034x; 1.3034x over previous
import functools

import jax
import jax.numpy as jnp
from jax import lax
from jax.experimental import pallas as pl
from jax.experimental.pallas import tpu as pltpu
from jax.experimental.pallas import tpu_sc as plsc

DIM = 64
B = 16384
LAM = 0.001

NC = 2
NS = 16
NW = NC * NS

SSPAN = 31232
W = 512
NCH = SSPAN // W
XBASE = 999424
TAIL0 = 999936
SCAP = 144
BCAP = 6144
DUMP = B

_i32 = jnp.int32
_f32 = jnp.float32


def _lane():
    return lax.iota(_i32, 16)


def _splat(x, dtype=_i32):
    return jnp.full((16,), x, dtype)


def _bin_ids(ids_hbm, idbuf, binref, lo, hi):
    lane = _lane()

    def chunk(ci, cnt):
        pltpu.sync_copy(ids_hbm.at[pl.ds(ci * 512, 512)], idbuf)

        def vec(jv, cnt):
            v = idbuf[pl.ds(jv * 16, 16)]
            mask = (v >= lo) & (v < hi)
            pos = ci * 512 + jv * 16 + lane
            packed = ((v - lo) << 14) | pos
            cum = plsc.cumsum(mask.astype(_i32))
            dest = jnp.minimum(cnt + cum - 1, BCAP - 1)
            plsc.store_scatter(binref, [dest], packed, mask=mask)
            return cnt + plsc.all_reduce_population_count(mask)[0]

        return lax.fori_loop(0, 32, vec, cnt)

    cnt = lax.fori_loop(0, 32, chunk, jnp.int32(0))
    return jnp.minimum(cnt, BCAP)


def _scan_chunk(buf, wc, c0l, binref, cnt, staged, posbuf, inter, semS, m):
    lane = _lane()
    c0 = jnp.minimum(jnp.asarray(c0l, _i32), 1 << 15)
    lo_p = c0 << 14
    hi_p = (c0 + wc) << 14
    nv = (cnt + 15) >> 4

    def vec(jv, m):
        idx16 = jv * 16 + lane
        v = binref[pl.ds(jv * 16, 16)]
        inr = (v >= lo_p) & (v < hi_p) & (idx16 < cnt)
        pci = plsc.all_reduce_population_count(inr)[0]
        cum = plsc.cumsum(inr.astype(_i32))
        dest = m + cum - 1
        cols = (v >> 14) - c0
        pos = v & 0x3FFF

        @pl.when(pci > 0)
        def _():
            def feat(j, carry):
                js = _splat(0) + j
                val = plsc.load_gather(buf, [js, cols], mask=inr)
                plsc.store_scatter(staged, [dest, js], val, mask=inr)
                return carry

            lax.fori_loop(0, DIM, feat, 0)
            plsc.store_scatter(posbuf, [dest >> 7, dest & 127], pos,
                               mask=inr)

        m2 = m + pci

        @pl.when(m2 >= 128)
        def _():
            pltpu.async_copy(staged.at[pl.ds(0, 128)],
                             inter.at[posbuf.at[0]], semS).wait()
            src = 128 + lane

            def feat(j, carry):
                js = _splat(0) + j
                sp = plsc.load_gather(staged, [src, js])
                plsc.store_scatter(staged, [lane, js], sp)
                return carry

            lax.fori_loop(0, DIM, feat, 0)
            pp = plsc.load_gather(posbuf, [_splat(1), lane])
            plsc.store_scatter(posbuf, [_splat(0), lane], pp)

        return jnp.where(m2 >= 128, m2 - 128, m2)

    return lax.fori_loop(0, nv, vec, m)


def _final_flush(staged, posbuf, inter, semS, m):
    lane = _lane()
    for v8 in range(8):
        idxv = v8 * 16 + lane
        plsc.store_scatter(posbuf, [_splat(0), idxv], _splat(DUMP),
                           mask=idxv >= m)
    pltpu.async_copy(staged.at[pl.ds(0, 128)],
                     inter.at[posbuf.at[0]], semS).wait()


def _sc1_body(ue_hbm, ie_hbm, tu_hbm, ti_hbm, uid_hbm, pid_hbm, nid_hbm,
              iu_hbm, ip_hbm, in_hbm,
              idbuf, ubin, pbin, nbin,
              buf0, buf1, tailbuf, stA, stB, pbA, pbB,
              sem0, sem1, semS):
    c = lax.axis_index("c")
    s = lax.axis_index("s")
    wid = s * NC + c
    lo = wid * SSPAN
    hi = jnp.where(wid == NW - 1, 1000000, lo + SSPAN)

    cu = _bin_ids(uid_hbm, idbuf, ubin, lo, hi)
    cp_ = _bin_ids(pid_hbm, idbuf, pbin, lo, hi)
    cn = _bin_ids(nid_hbm, idbuf, nbin, lo, hi)

    def sweep(table, tail, lists, ms):
        def scan_all(buf, wc, c0l, ms):
            return tuple(
                _scan_chunk(buf, wc, c0l, b, ct, st, pb, it, semS, m)
                for (b, ct, st, pb, it), m in zip(lists, ms))

        def start(buf, sem, ci):
            return pltpu.async_copy(
                table.at[:, pl.ds(lo + ci * W, W)], buf, sem)

        start(buf0, sem0, 0)
        start(buf1, sem1, 1)

        def pair(kp, ms):
            pltpu.make_async_copy(
                table.at[:, pl.ds(lo, W)], buf0, sem0).wait()
            ms = scan_all(buf0, W, (2 * kp) * W, ms)
            start(buf0, sem0, 2 * kp + 2)

            pltpu.make_async_copy(
                table.at[:, pl.ds(lo, W)], buf1, sem1).wait()
            ms = scan_all(buf1, W, (2 * kp + 1) * W, ms)

            @pl.when(kp < NCH // 2 - 1)
            def _():
                start(buf1, sem1, 2 * kp + 3)

            return ms

        ms = lax.fori_loop(0, NCH // 2, pair, tuple(ms))
        pltpu.make_async_copy(
            table.at[:, pl.ds(lo, W)], buf0, sem0).wait()
        ms = scan_all(buf0, W, (NCH - 1) * W, ms)

        pltpu.sync_copy(table.at[:, pl.ds(XBASE, W)], buf0)
        ms = scan_all(buf0, W, jnp.int32(XBASE) - lo, ms)
        pltpu.sync_copy(tail, tailbuf)
        ms = scan_all(tailbuf, 64, jnp.int32(TAIL0) - lo, ms)
        return ms

    z = jnp.int32(0)
    (mu,) = sweep(ue_hbm, tu_hbm, [(ubin, cu, stA, pbA, iu_hbm)], (z,))
    _final_flush(stA, pbA, iu_hbm, semS, mu)
    mp, mn = sweep(ie_hbm, ti_hbm, [(pbin, cp_, stA, pbA, ip_hbm),
                                    (nbin, cn, stB, pbB, in_hbm)], (z, z))
    _final_flush(stA, pbA, ip_hbm, semS, mp)
    _final_flush(stB, pbB, in_hbm, semS, mn)


HALF = 256


def _sc2_body(iu_hbm, ip_hbm, in_hbm, diff_hbm, l2_hbm,
              ubuf, pbuf, nbuf, diff_v, l2_v, sem):
    c = lax.axis_index("c")
    s = lax.axis_index("s")
    wid = s * NC + c

    lane = _lane()
    zero = jnp.zeros((16,), _f32)

    for h in range(2):
        base = wid * (2 * HALF) + h * HALF
        cps = [pltpu.async_copy(iu_hbm.at[pl.ds(base, HALF)], ubuf, sem),
               pltpu.async_copy(ip_hbm.at[pl.ds(base, HALF)], pbuf, sem),
               pltpu.async_copy(in_hbm.at[pl.ds(base, HALF)], nbuf, sem)]
        for cp in cps:
            cp.wait()

        def group(g, carry):
            bvec = g * 16 + lane

            def feat(j, acc):
                pos, neg, l2 = acc
                js = _splat(0) + j
                u = plsc.load_gather(ubuf, [bvec, js])
                p = plsc.load_gather(pbuf, [bvec, js])
                n = plsc.load_gather(nbuf, [bvec, js])
                return (pos + u * p, neg + u * n,
                        l2 + (u * u + (p * p + n * n)))

            pos, neg, l2 = lax.fori_loop(0, DIM, feat, (zero, zero, zero))
            off = h * HALF + g * 16
            diff_v[pl.ds(off, 16)] = neg - pos
            l2_v[pl.ds(off, 16)] = 0.5 * l2
            return carry

        lax.fori_loop(0, HALF // 16, group, 0)

    pltpu.sync_copy(diff_v, diff_hbm.at[pl.ds(wid * 2 * HALF, 2 * HALF)])
    pltpu.sync_copy(l2_v, l2_hbm.at[pl.ds(wid * 2 * HALF, 2 * HALF)])


def _tc_tail_body(ue_ref, ie_ref, tu_ref, ti_ref):
    tu_ref[...] = ue_ref[...]
    ti_ref[...] = ie_ref[...]


def _tc_body(diff_ref, l2_ref, out_ref):
    x = diff_ref[:]
    sp = jnp.maximum(x, 0.0) + jnp.log1p(jnp.exp(-jnp.abs(x)))
    out_ref[0, 0] = jnp.sum(sp) / B + LAM * (jnp.sum(l2_ref[:]) / B)


def kernel(user_embed, item_embed, user_ids, item_pos_ids, item_neg_ids):
    uid = user_ids.astype(_i32)
    pid = item_pos_ids.astype(_i32)
    nid = item_neg_ids.astype(_i32)

    mesh = plsc.VectorSubcoreMesh(core_axis_name="c", subcore_axis_name="s")
    params = pltpu.CompilerParams(needs_layout_passes=False)

    ueT = user_embed.T
    ieT = item_embed.T
    tail_spec = pl.BlockSpec((DIM, 128), lambda i: (0, TAIL0 // 128))
    out_spec = pl.BlockSpec((DIM, 128), lambda i: (0, 0))
    tu, ti = pl.pallas_call(
        _tc_tail_body,
        grid=(1,),
        out_shape=[jax.ShapeDtypeStruct((DIM, 128), _f32),
                   jax.ShapeDtypeStruct((DIM, 128), _f32)],
        in_specs=[tail_spec, tail_spec],
        out_specs=[out_spec, out_spec],
    )(ueT, ieT)

    sc1 = functools.partial(
        pl.kernel,
        mesh=mesh,
        compiler_params=params,
        out_type=[
            jax.ShapeDtypeStruct((B + 16, 128), _f32),
            jax.ShapeDtypeStruct((B + 16, 128), _f32),
            jax.ShapeDtypeStruct((B + 16, 128), _f32),
        ],
        scratch_types=[
            pltpu.VMEM((512,), _i32),
            pltpu.VMEM((BCAP,), _i32),
            pltpu.VMEM((BCAP,), _i32),
            pltpu.VMEM((BCAP,), _i32),
            pltpu.VMEM((DIM, W), _f32),
            pltpu.VMEM((DIM, W), _f32),
            pltpu.VMEM((DIM, 128), _f32),
            pltpu.VMEM((SCAP, 128), _f32),
            pltpu.VMEM((SCAP, 128), _f32),
            pltpu.VMEM((2, 128), _i32),
            pltpu.VMEM((2, 128), _i32),
            pltpu.SemaphoreType.DMA,
            pltpu.SemaphoreType.DMA,
            pltpu.SemaphoreType.DMA,
        ],
    )(_sc1_body)
    iu, ip_, in_ = sc1(ueT, ieT, tu, ti, uid, pid, nid)

    sc2 = functools.partial(
        pl.kernel,
        mesh=mesh,
        compiler_params=params,
        out_type=[
            jax.ShapeDtypeStruct((B,), _f32),
            jax.ShapeDtypeStruct((B,), _f32),
        ],
        scratch_types=[
            pltpu.VMEM((HALF, 128), _f32),
            pltpu.VMEM((HALF, 128), _f32),
            pltpu.VMEM((HALF, 128), _f32),
            pltpu.VMEM((2 * HALF,), _f32),
            pltpu.VMEM((2 * HALF,), _f32),
            pltpu.SemaphoreType.DMA,
        ],
    )(_sc2_body)
    diff, l2row = sc2(iu, ip_, in_)

    out = pl.pallas_call(
        _tc_body,
        out_shape=jax.ShapeDtypeStruct((1, 1), _f32),
        out_specs=pl.BlockSpec(memory_space=pltpu.SMEM),
    )(diff.reshape(B // 128, 128), l2row.reshape(B // 128, 128))
    return out[0, 0]


# speedup vs baseline: 1.7769x; 1.3633x over previous
import functools

import jax
import jax.numpy as jnp
from jax import lax
from jax.experimental import pallas as pl
from jax.experimental.pallas import tpu as pltpu
from jax.experimental.pallas import tpu_sc as plsc

DIM = 64
B = 16384
LAM = 0.001

NC = 2
NS = 16
NW = NC * NS

SSPAN = 31232
W = 512
NCH = SSPAN // W
XBASE = 999424
TAIL0 = 999936
SCAP = 144
BCAP = 6144
DUMP = B

_i32 = jnp.int32
_f32 = jnp.float32


def _lane():
    return lax.iota(_i32, 16)


def _splat(x, dtype=_i32):
    return jnp.full((16,), x, dtype)


def _bin_ids(ids_hbm, idbuf, binref, lo, hi):
    lane = _lane()

    def chunk(ci, cnt):
        pltpu.sync_copy(ids_hbm.at[pl.ds(ci * 512, 512)], idbuf)

        def vec(jv, cnt):
            v = idbuf[pl.ds(jv * 16, 16)]
            mask = (v >= lo) & (v < hi)
            pos = ci * 512 + jv * 16 + lane
            packed = ((v - lo) << 14) | pos
            cum = plsc.cumsum(mask.astype(_i32))
            dest = jnp.minimum(cnt + cum - 1, BCAP - 1)
            plsc.store_scatter(binref, [dest], packed, mask=mask)
            return cnt + plsc.all_reduce_population_count(mask)[0]

        return lax.fori_loop(0, 32, vec, cnt)

    cnt = lax.fori_loop(0, 32, chunk, jnp.int32(0))
    return jnp.minimum(cnt, BCAP)


def _scan_chunk(buf, wc, c0l, binref, cnt, staged, posbuf, mvec, inter,
                semS, m):
    lane = _lane()
    c0 = jnp.minimum(jnp.asarray(c0l, _i32), 1 << 15)
    lo_p = c0 << 14
    hi_p = (c0 + wc) << 14
    nv = (cnt + 15) >> 4

    def vec(jv, mc):
        idx16 = jv * 16 + lane
        v = binref[pl.ds(jv * 16, 16)]
        inr = (v >= lo_p) & (v < hi_p) & (idx16 < cnt)
        pci = plsc.all_reduce_population_count(inr)[0]

        @pl.when(pci > 0)
        def _():
            cum = plsc.cumsum(inr.astype(_i32))
            mdest = mc + cum - 1
            packed = ((v >> 14) - c0) << 14 | (v & 0x3FFF)
            plsc.store_scatter(mvec, [mdest], packed, mask=inr)

        return mc + pci

    def seg(sg, m):
        v0 = sg * 32
        mc = lax.fori_loop(v0, jnp.minimum(nv, v0 + 32), vec, jnp.int32(0))

        def grp_m(g, m):
            ge = g * 16 + lane
            valid = ge < mc
            mv = mvec[pl.ds(g * 16, 16)]
            cols = mv >> 14
            pos = mv & 0x3FFF
            dest = m + lane

            def featg(j, carry):
                js = _splat(0) + j
                val = plsc.load_gather(buf, [js, cols], mask=valid)
                plsc.store_scatter(staged, [dest, js], val, mask=valid)
                return carry

            lax.fori_loop(0, DIM, featg, 0)
            plsc.store_scatter(posbuf, [dest >> 7, dest & 127], pos,
                               mask=valid)
            pcg = jnp.minimum(mc - g * 16, 16)
            m2 = m + pcg

            @pl.when(m2 >= 128)
            def _():
                pltpu.async_copy(staged.at[pl.ds(0, 128)],
                                 inter.at[posbuf.at[0]], semS).wait()
                src = 128 + lane

                def feat(j, carry):
                    js = _splat(0) + j
                    sp = plsc.load_gather(staged, [src, js])
                    plsc.store_scatter(staged, [lane, js], sp)
                    return carry

                lax.fori_loop(0, DIM, feat, 0)
                pp = plsc.load_gather(posbuf, [_splat(1), lane])
                plsc.store_scatter(posbuf, [_splat(0), lane], pp)

            return jnp.where(m2 >= 128, m2 - 128, m2)

        return lax.fori_loop(0, (mc + 15) >> 4, grp_m, m)

    return lax.fori_loop(0, (nv + 31) >> 5, seg, m)


def _final_flush(staged, posbuf, inter, semS, m):
    lane = _lane()
    for v8 in range(8):
        idxv = v8 * 16 + lane
        plsc.store_scatter(posbuf, [_splat(0), idxv], _splat(DUMP),
                           mask=idxv >= m)
    pltpu.async_copy(staged.at[pl.ds(0, 128)],
                     inter.at[posbuf.at[0]], semS).wait()


def _sc1_body(ue_hbm, ie_hbm, tu_hbm, ti_hbm, uid_hbm, pid_hbm, nid_hbm,
              iu_hbm, ip_hbm, in_hbm,
              idbuf, ubin, pbin, nbin,
              buf0, buf1, tailbuf, stA, stB, pbA, pbB, mvec,
              sem0, sem1, semS):
    c = lax.axis_index("c")
    s = lax.axis_index("s")
    wid = s * NC + c
    lo = wid * SSPAN
    hi = jnp.where(wid == NW - 1, 1000000, lo + SSPAN)

    cu = _bin_ids(uid_hbm, idbuf, ubin, lo, hi)
    cp_ = _bin_ids(pid_hbm, idbuf, pbin, lo, hi)
    cn = _bin_ids(nid_hbm, idbuf, nbin, lo, hi)

    def sweep(table, tail, lists, ms):
        def scan_all(buf, wc, c0l, ms):
            return tuple(
                _scan_chunk(buf, wc, c0l, b, ct, st, pb, mvec, it, semS, m)
                for (b, ct, st, pb, it), m in zip(lists, ms))

        def start(buf, sem, ci):
            return pltpu.async_copy(
                table.at[:, pl.ds(lo + ci * W, W)], buf, sem)

        start(buf0, sem0, 0)
        start(buf1, sem1, 1)

        def pair(kp, ms):
            pltpu.make_async_copy(
                table.at[:, pl.ds(lo, W)], buf0, sem0).wait()
            ms = scan_all(buf0, W, (2 * kp) * W, ms)
            start(buf0, sem0, 2 * kp + 2)

            pltpu.make_async_copy(
                table.at[:, pl.ds(lo, W)], buf1, sem1).wait()
            ms = scan_all(buf1, W, (2 * kp + 1) * W, ms)

            @pl.when(kp < NCH // 2 - 1)
            def _():
                start(buf1, sem1, 2 * kp + 3)

            return ms

        ms = lax.fori_loop(0, NCH // 2, pair, tuple(ms))
        pltpu.make_async_copy(
            table.at[:, pl.ds(lo, W)], buf0, sem0).wait()
        ms = scan_all(buf0, W, (NCH - 1) * W, ms)

        pltpu.sync_copy(table.at[:, pl.ds(XBASE, W)], buf0)
        ms = scan_all(buf0, W, jnp.int32(XBASE) - lo, ms)
        pltpu.sync_copy(tail, tailbuf)
        ms = scan_all(tailbuf, 64, jnp.int32(TAIL0) - lo, ms)
        return ms

    z = jnp.int32(0)
    (mu,) = sweep(ue_hbm, tu_hbm, [(ubin, cu, stA, pbA, iu_hbm)], (z,))
    _final_flush(stA, pbA, iu_hbm, semS, mu)
    mp, mn = sweep(ie_hbm, ti_hbm, [(pbin, cp_, stA, pbA, ip_hbm),
                                    (nbin, cn, stB, pbB, in_hbm)], (z, z))
    _final_flush(stA, pbA, ip_hbm, semS, mp)
    _final_flush(stB, pbB, in_hbm, semS, mn)


HALF = 256


def _sc2_body(iu_hbm, ip_hbm, in_hbm, diff_hbm, l2_hbm,
              ubuf, pbuf, nbuf, diff_v, l2_v, sem):
    c = lax.axis_index("c")
    s = lax.axis_index("s")
    wid = s * NC + c

    lane = _lane()
    zero = jnp.zeros((16,), _f32)

    for h in range(2):
        base = wid * (2 * HALF) + h * HALF
        cps = [pltpu.async_copy(iu_hbm.at[pl.ds(base, HALF)], ubuf, sem),
               pltpu.async_copy(ip_hbm.at[pl.ds(base, HALF)], pbuf, sem),
               pltpu.async_copy(in_hbm.at[pl.ds(base, HALF)], nbuf, sem)]
        for cp in cps:
            cp.wait()

        def group(g, carry):
            bvec = g * 16 + lane

            def feat(j, acc):
                pos, neg, l2 = acc
                js = _splat(0) + j
                u = plsc.load_gather(ubuf, [bvec, js])
                p = plsc.load_gather(pbuf, [bvec, js])
                n = plsc.load_gather(nbuf, [bvec, js])
                return (pos + u * p, neg + u * n,
                        l2 + (u * u + (p * p + n * n)))

            pos, neg, l2 = lax.fori_loop(0, DIM, feat, (zero, zero, zero))
            off = h * HALF + g * 16
            diff_v[pl.ds(off, 16)] = neg - pos
            l2_v[pl.ds(off, 16)] = 0.5 * l2
            return carry

        lax.fori_loop(0, HALF // 16, group, 0)

    pltpu.sync_copy(diff_v, diff_hbm.at[pl.ds(wid * 2 * HALF, 2 * HALF)])
    pltpu.sync_copy(l2_v, l2_hbm.at[pl.ds(wid * 2 * HALF, 2 * HALF)])


def _tc_tail_body(ue_ref, ie_ref, tu_ref, ti_ref):
    tu_ref[...] = ue_ref[...]
    ti_ref[...] = ie_ref[...]


def _tc_body(diff_ref, l2_ref, out_ref):
    x = diff_ref[:]
    sp = jnp.maximum(x, 0.0) + jnp.log1p(jnp.exp(-jnp.abs(x)))
    out_ref[0, 0] = jnp.sum(sp) / B + LAM * (jnp.sum(l2_ref[:]) / B)


def kernel(user_embed, item_embed, user_ids, item_pos_ids, item_neg_ids):
    uid = user_ids.astype(_i32)
    pid = item_pos_ids.astype(_i32)
    nid = item_neg_ids.astype(_i32)

    mesh = plsc.VectorSubcoreMesh(core_axis_name="c", subcore_axis_name="s")
    params = pltpu.CompilerParams(needs_layout_passes=False)

    ueT = user_embed.T
    ieT = item_embed.T
    tail_spec = pl.BlockSpec((DIM, 128), lambda i: (0, TAIL0 // 128))
    out_spec = pl.BlockSpec((DIM, 128), lambda i: (0, 0))
    tu, ti = pl.pallas_call(
        _tc_tail_body,
        grid=(1,),
        out_shape=[jax.ShapeDtypeStruct((DIM, 128), _f32),
                   jax.ShapeDtypeStruct((DIM, 128), _f32)],
        in_specs=[tail_spec, tail_spec],
        out_specs=[out_spec, out_spec],
    )(ueT, ieT)

    sc1 = functools.partial(
        pl.kernel,
        mesh=mesh,
        compiler_params=params,
        out_type=[
            jax.ShapeDtypeStruct((B + 16, 128), _f32),
            jax.ShapeDtypeStruct((B + 16, 128), _f32),
            jax.ShapeDtypeStruct((B + 16, 128), _f32),
        ],
        scratch_types=[
            pltpu.VMEM((512,), _i32),
            pltpu.VMEM((BCAP,), _i32),
            pltpu.VMEM((BCAP,), _i32),
            pltpu.VMEM((BCAP,), _i32),
            pltpu.VMEM((DIM, W), _f32),
            pltpu.VMEM((DIM, W), _f32),
            pltpu.VMEM((DIM, 128), _f32),
            pltpu.VMEM((SCAP, 128), _f32),
            pltpu.VMEM((SCAP, 128), _f32),
            pltpu.VMEM((2, 128), _i32),
            pltpu.VMEM((2, 128), _i32),
            pltpu.VMEM((512,), _i32),
            pltpu.SemaphoreType.DMA,
            pltpu.SemaphoreType.DMA,
            pltpu.SemaphoreType.DMA,
        ],
    )(_sc1_body)
    iu, ip_, in_ = sc1(ueT, ieT, tu, ti, uid, pid, nid)

    sc2 = functools.partial(
        pl.kernel,
        mesh=mesh,
        compiler_params=params,
        out_type=[
            jax.ShapeDtypeStruct((B,), _f32),
            jax.ShapeDtypeStruct((B,), _f32),
        ],
        scratch_types=[
            pltpu.VMEM((HALF, 128), _f32),
            pltpu.VMEM((HALF, 128), _f32),
            pltpu.VMEM((HALF, 128), _f32),
            pltpu.VMEM((2 * HALF,), _f32),
            pltpu.VMEM((2 * HALF,), _f32),
            pltpu.SemaphoreType.DMA,
        ],
    )(_sc2_body)
    diff, l2row = sc2(iu, ip_, in_)

    out = pl.pallas_call(
        _tc_body,
        out_shape=jax.ShapeDtypeStruct((1, 1), _f32),
        out_specs=pl.BlockSpec(memory_space=pltpu.SMEM),
    )(diff.reshape(B // 128, 128), l2row.reshape(B // 128, 128))
    return out[0, 0]
